# baseline (speedup 1.0000x reference)
import functools

import jax
import jax.numpy as jnp
from jax import lax
from jax.experimental import pallas as pl
from jax.experimental.pallas import tpu as pltpu
from jax.experimental.pallas import tpu_sc as plsc

D = 64
L = 16
GW = 128
CHUNK = 256
NBUF = 3


def _rsqrt(x):
    i = plsc.bitcast(x, jnp.int32)
    i = jnp.int32(0x5F3759DF) - (i >> 1)
    y = plsc.bitcast(i, jnp.float32)
    h = x * jnp.float32(0.5)
    for _ in range(2):
        y = y * (jnp.float32(1.5) - h * y * y)
    return y


@functools.partial(jax.jit, static_argnames=("b_total",))
def _embed_normalize(x_flat, table, b_total):
    info = plsc.get_sparse_core_info()
    nc, ns = info.num_cores, info.num_subcores
    nw = nc * ns
    b_per_w = b_total // nw
    n_chunks = b_per_w // CHUNK
    mesh = plsc.VectorSubcoreMesh(core_axis_name="c", subcore_axis_name="s")

    @functools.partial(
        pl.kernel,
        mesh=mesh,
        out_type=jax.ShapeDtypeStruct((b_total, D), jnp.float32),
        compiler_params=pltpu.CompilerParams(
            needs_layout_passes=False, use_tc_tiling_on_sc=False),
        scratch_types=[
            pltpu.VMEM((NBUF, CHUNK), jnp.int32),
            pltpu.VMEM((NBUF, CHUNK, D), jnp.float32),
            pltpu.SemaphoreType.DMA((NBUF,)),
            pltpu.SemaphoreType.DMA((NBUF,)),
            pltpu.SemaphoreType.DMA((NBUF,)),
        ],
    )
    def body(x_hbm, table_hbm, out_hbm, idx_v, rows_v, sem_i, sem_g, sem_o):
        wid = lax.axis_index("s") * nc + lax.axis_index("c")
        base = wid * b_per_w

        def slot(g):
            return lax.rem(g, NBUF)

        def idx_dma(g):
            b = slot(g)
            return pltpu.make_async_copy(
                x_hbm.at[pl.ds(base + g * CHUNK, CHUNK)],
                idx_v.at[b], sem_i.at[b])

        def gather_dma(g, j):
            b = slot(g)
            return pltpu.make_async_copy(
                table_hbm.at[idx_v.at[b, pl.ds(j * GW, GW)]],
                rows_v.at[b, pl.ds(j * GW, GW)], sem_g.at[b])

        def out_dma(g):
            b = slot(g)
            return pltpu.make_async_copy(
                rows_v.at[b],
                out_hbm.at[pl.ds(base + g * CHUNK, CHUNK)], sem_o.at[b])

        idx_dma(0).start()
        idx_dma(1).start()
        idx_dma(0).wait()
        for j in range(CHUNK // GW):
            gather_dma(0, j).start()

        def chunk_body(g, carry):
            b = slot(g)
            @pl.when(g + 1 < n_chunks)
            def _():
                @pl.when(g + 1 >= NBUF)
                def _():
                    out_dma(g + 1 - NBUF).wait()
                idx_dma(g + 1).wait()
                for j in range(CHUNK // GW):
                    gather_dma(g + 1, j).start()

            @pl.when(g + 2 < n_chunks)
            def _():
                idx_dma(g + 2).start()

            for j in range(CHUNK // GW):
                gather_dma(g, j).wait()

            def grp(t, c):
                row0 = t * L
                for r in range(L):
                    vs = [rows_v[b, row0 + r, pl.ds(q * L, L)]
                          for q in range(D // L)]
                    acc = None
                    for v in vs:
                        acc = v * v if acc is None else acc + v * v
                    sv = jnp.full((L,), jnp.sum(acc), jnp.float32)
                    scale = _rsqrt(sv)
                    for q, v in enumerate(vs):
                        rows_v[b, row0 + r, pl.ds(q * L, L)] = v * scale
                return c

            lax.fori_loop(0, CHUNK // L, grp, 0)
            out_dma(g).start()
            return carry

        lax.fori_loop(0, n_chunks, chunk_body, 0)
        for t in range(NBUF):
            out_dma(n_chunks - 1 - t).wait()

    return body(x_flat, table)


def kernel(x, table):
    b, h = x.shape
    out = _embed_normalize(x.reshape(-1), table, b * h)
    return out.reshape(b, h, D)
